# speedup vs baseline: 1.1908x; 1.1908x over previous
import functools
import jax
import jax.numpy as jnp
from jax import lax
from jax.experimental import pallas as pl
from jax.experimental.pallas import tpu as pltpu

_D = 128
_INV_SQRT2 = 0.7071067811865476


def _gelu(x):
    return 0.5 * x * (1.0 + lax.erf(x * _INV_SQRT2))


def _edge_mlp_body(absd, gsum, nd, b1, b2, w2, eb1, eb2, out):
    h1 = (jnp.dot(absd[...], b1[...], preferred_element_type=jnp.float32)
          + jnp.dot(nd[...], b2[...], preferred_element_type=jnp.float32)
          + gsum[...] + eb1[...])
    h2 = jnp.dot(_gelu(h1), w2[...], preferred_element_type=jnp.float32) + eb2[...]
    out[...] = _gelu(h2)


def _edge_mlp(absd, gsum, nd, b1, b2, w2, eb1, eb2, blk=2000):
    e = absd.shape[0]
    grid = e // blk
    row = lambda i: (i, 0)
    full = lambda i: (0, 0)
    return pl.pallas_call(
        _edge_mlp_body,
        grid=(grid,),
        in_specs=[
            pl.BlockSpec((blk, _D), row),
            pl.BlockSpec((blk, _D), row),
            pl.BlockSpec((blk, _D), row),
            pl.BlockSpec((_D, _D), full),
            pl.BlockSpec((_D, _D), full),
            pl.BlockSpec((_D, _D), full),
            pl.BlockSpec((1, _D), full),
            pl.BlockSpec((1, _D), full),
        ],
        out_specs=pl.BlockSpec((blk, _D), row),
        out_shape=jax.ShapeDtypeStruct((e, _D), jnp.float32),
    )(absd, gsum, nd, b1, b2, w2, eb1.reshape(1, _D), eb2.reshape(1, _D))


def _node_mlp_body(x, nodes, s, rcc, w1, w2, nb1, nb2, out):
    agg = s[...] * rcc[...]
    cat = jnp.concatenate([x[...], nodes[...], agg], axis=1)
    h1 = jnp.dot(cat, w1[...], preferred_element_type=jnp.float32) + nb1[...]
    h2 = jnp.dot(_gelu(h1), w2[...], preferred_element_type=jnp.float32) + nb2[...]
    out[...] = _gelu(h2)


def _node_mlp(x, nodes, s, rcc, w1, w2, nb1, nb2, blk=2000):
    n = x.shape[0]
    grid = n // blk
    row = lambda i: (i, 0)
    full = lambda i: (0, 0)
    return pl.pallas_call(
        _node_mlp_body,
        grid=(grid,),
        in_specs=[
            pl.BlockSpec((blk, _D), row),
            pl.BlockSpec((blk, _D), row),
            pl.BlockSpec((blk, _D), row),
            pl.BlockSpec((blk, 1), row),
            pl.BlockSpec((3 * _D, _D), full),
            pl.BlockSpec((_D, _D), full),
            pl.BlockSpec((1, _D), full),
            pl.BlockSpec((1, _D), full),
        ],
        out_specs=pl.BlockSpec((blk, _D), row),
        out_shape=jax.ShapeDtypeStruct((n, _D), jnp.float32),
    )(x, nodes, s, rcc, w1, w2, nb1.reshape(1, _D), nb2.reshape(1, _D))


def _finish_body(xu, t, c, rcc, nodes, a1, a2, cw, x_out, g_out):
    xn = 0.5 * (c[...] * xu[...] + t[...]) * rcc[...]
    x_out[...] = xn
    g_out[...] = (jnp.dot(xn, a1[...], preferred_element_type=jnp.float32)
                  + jnp.dot(nodes[...], a2[...], preferred_element_type=jnp.float32)
                  + jnp.dot(xu[...], cw[...], preferred_element_type=jnp.float32))


def _finish(xu, t, c, rcc, nodes, a1, a2, cw, blk=2000):
    n = xu.shape[0]
    grid = n // blk
    row = lambda i: (i, 0)
    full = lambda i: (0, 0)
    return pl.pallas_call(
        _finish_body,
        grid=(grid,),
        in_specs=[
            pl.BlockSpec((blk, _D), row),
            pl.BlockSpec((blk, _D), row),
            pl.BlockSpec((blk, 1), row),
            pl.BlockSpec((blk, 1), row),
            pl.BlockSpec((blk, _D), row),
            pl.BlockSpec((_D, _D), full),
            pl.BlockSpec((_D, _D), full),
            pl.BlockSpec((_D, _D), full),
        ],
        out_specs=[pl.BlockSpec((blk, _D), row), pl.BlockSpec((blk, _D), row)],
        out_shape=[jax.ShapeDtypeStruct((n, _D), jnp.float32),
                   jax.ShapeDtypeStruct((n, _D), jnp.float32)],
    )(xu, t, c, rcc, nodes, a1, a2, cw)


def _decode_body(xu, t, c, rcc, w, b, out):
    xn = 0.5 * (c[...] * xu[...] + t[...]) * rcc[...]
    out[...] = jnp.dot(xn, w[...], preferred_element_type=jnp.float32) + b[...]


def _decode(xu, t, c, rcc, w, b, blk=2000):
    n = xu.shape[0]
    kout = w.shape[1]
    grid = n // blk
    row = lambda i: (i, 0)
    full = lambda i: (0, 0)
    return pl.pallas_call(
        _decode_body,
        grid=(grid,),
        in_specs=[
            pl.BlockSpec((blk, _D), row),
            pl.BlockSpec((blk, _D), row),
            pl.BlockSpec((blk, 1), row),
            pl.BlockSpec((blk, 1), row),
            pl.BlockSpec((_D, kout), full),
            pl.BlockSpec((1, kout), full),
        ],
        out_specs=pl.BlockSpec((blk, kout), row),
        out_shape=jax.ShapeDtypeStruct((n, kout), jnp.float32),
    )(xu, t, c, rcc, w, b.reshape(1, kout))


def _tables0_body(nodes, w, out):
    out[...] = jnp.dot(nodes[...], w[...], preferred_element_type=jnp.float32)


def _tables0(nodes, w, blk=2000):
    n = nodes.shape[0]
    grid = n // blk
    return pl.pallas_call(
        _tables0_body,
        grid=(grid,),
        in_specs=[pl.BlockSpec((blk, _D), lambda i: (i, 0)),
                  pl.BlockSpec((_D, _D), lambda i: (0, 0))],
        out_specs=pl.BlockSpec((blk, _D), lambda i: (i, 0)),
        out_shape=jax.ShapeDtypeStruct((n, _D), jnp.float32),
    )(nodes, w)


def kernel(nodes, params, edge_index):
    n, d = nodes.shape
    L = 3
    src = edge_index[0].astype(jnp.int32)
    dst = edge_index[1].astype(jnp.int32)

    c = jax.ops.segment_sum(jnp.ones(src.shape, jnp.float32), src, num_segments=n)
    rcc = (1.0 / jnp.clip(c, 1.0, None)).reshape(n, 1)
    ccol = c.reshape(n, 1)

    ND = 0.5 * jnp.abs(nodes[dst] - nodes[src])

    x = nodes
    eW1_0 = params[0]
    g = _tables0(nodes, eW1_0[:d] + eW1_0[d:2*d] + eW1_0[4*d:])

    out = None
    for l in range(L):
        eW1, eb1, eW2, eb2, nW1, nb1, nW2, nb2 = params[8 * l:8 * l + 8]
        B1 = eW1[2*d:3*d]
        B2 = eW1[3*d:4*d]

        absd = 0.5 * jnp.abs(x[dst] - x[src])
        gsum = 0.5 * (g[dst] + g[src])
        ue = _edge_mlp(absd, gsum, ND, B1, B2, eW2, eb1, eb2)

        s = jax.ops.segment_sum(ue, src, num_segments=n)
        xu = _node_mlp(x, nodes, s, rcc, nW1, nW2, nb1, nb2)

        t = jax.ops.segment_sum(xu[dst], src, num_segments=n)
        if l + 1 < L:
            eW1n = params[8 * (l + 1)]
            x, g = _finish(xu, t, ccol, rcc, nodes,
                           eW1n[:d], eW1n[d:2*d], eW1n[4*d:])
        else:
            out = _decode(xu, t, ccol, rcc, params[8 * L], params[8 * L + 1])
    return out


# speedup vs baseline: 3.9724x; 3.3360x over previous
import functools
import jax
import jax.numpy as jnp
from jax import lax
from jax.experimental import pallas as pl
from jax.experimental.pallas import tpu as pltpu
from jax.experimental.pallas import tpu_sc as plsc

_D = 128
_NC = 2
_NS = 16
_NW = _NC * _NS
_K = 80
_INV_SQRT2 = 0.7071067811865476


def _gelu(x):
    return 0.5 * x * (1.0 + lax.erf(x * _INV_SQRT2))


def _mesh():
    return plsc.VectorSubcoreMesh(core_axis_name="c", subcore_axis_name="s")


def _gather2(tbl, dstx, srcx):
    e = dstx.shape[0]
    w = tbl.shape[1]
    per_w = e // _NW
    steps = per_w // _K

    @functools.partial(
        pl.kernel, mesh=_mesh(),
        out_type=[jax.ShapeDtypeStruct((e, w), jnp.float32),
                  jax.ShapeDtypeStruct((e, w), jnp.float32)],
        scratch_types=[pltpu.VMEM((_K,), jnp.int32),
                       pltpu.VMEM((_K,), jnp.int32),
                       pltpu.VMEM((_K, w), jnp.float32),
                       pltpu.VMEM((_K, w), jnp.float32),
                       pltpu.SemaphoreType.DMA,
                       pltpu.SemaphoreType.DMA])
    def kfn(tbl_h, dst_h, src_h, outd_h, outs_h, idxd, idxs, bufd, bufs,
            semd, sems):
        wid = lax.axis_index("s") * _NC + lax.axis_index("c")
        base = wid * per_w

        def body(j, carry):
            off = base + j * _K
            pltpu.sync_copy(dst_h.at[pl.ds(off, _K)], idxd)
            pltpu.sync_copy(src_h.at[pl.ds(off, _K)], idxs)
            cd = pltpu.async_copy(tbl_h.at[idxd], bufd, semd)
            cs = pltpu.async_copy(tbl_h.at[idxs], bufs, sems)
            cd.wait()
            cs.wait()
            pltpu.sync_copy(bufd, outd_h.at[pl.ds(off, _K)])
            pltpu.sync_copy(bufs, outs_h.at[pl.ds(off, _K)])
            return carry

        lax.fori_loop(0, steps, body, 0)

    return kfn(tbl, dstx, srcx)


def _scatter_sum(vals, srcx, zeros):
    e, w = vals.shape
    n = zeros.shape[0]
    per_w = e // _NW
    steps = per_w // _K
    rows_ps = n // _NS

    @functools.partial(
        pl.kernel, mesh=_mesh(),
        out_type=jax.ShapeDtypeStruct((_NC, n, w), jnp.float32),
        scratch_types=[pltpu.VMEM((_K,), jnp.int32),
                       pltpu.VMEM((_K, w), jnp.float32),
                       pltpu.VMEM_SHARED((n, w), jnp.float32)])
    def kfn(vals_h, src_h, zeros_h, out_h, idxv, buf, shared):
        cid = lax.axis_index("c")
        sid = lax.axis_index("s")
        wid = sid * _NC + cid
        row0 = sid * rows_ps
        pltpu.sync_copy(zeros_h.at[pl.ds(row0, rows_ps)],
                        shared.at[pl.ds(row0, rows_ps)])
        plsc.subcore_barrier()

        def body(j, carry):
            off = wid * per_w + j * _K
            pltpu.sync_copy(src_h.at[pl.ds(off, _K)], idxv)
            pltpu.sync_copy(vals_h.at[pl.ds(off, _K)], buf)
            pltpu.sync_copy(buf, shared.at[idxv], add=True)
            return carry

        lax.fori_loop(0, steps, body, 0)
        plsc.subcore_barrier()
        pltpu.sync_copy(shared.at[pl.ds(row0, rows_ps)],
                        out_h.at[cid, pl.ds(row0, rows_ps)])

    return kfn(vals, srcx, zeros)


def _smooth_scatter(xu, dstx, srcx, zeros):
    n, w = zeros.shape
    e = dstx.shape[0]
    per_w = e // _NW
    steps = per_w // _K
    rows_ps = n // _NS

    @functools.partial(
        pl.kernel, mesh=_mesh(),
        out_type=jax.ShapeDtypeStruct((_NC, n, w), jnp.float32),
        scratch_types=[pltpu.VMEM((_K,), jnp.int32),
                       pltpu.VMEM((_K,), jnp.int32),
                       pltpu.VMEM((_K, w), jnp.float32),
                       pltpu.VMEM_SHARED((n, w), jnp.float32),
                       pltpu.SemaphoreType.DMA])
    def kfn(xu_h, dst_h, src_h, zeros_h, out_h, idxd, idxs, buf, shared, sem):
        cid = lax.axis_index("c")
        sid = lax.axis_index("s")
        wid = sid * _NC + cid
        row0 = sid * rows_ps
        pltpu.sync_copy(zeros_h.at[pl.ds(row0, rows_ps)],
                        shared.at[pl.ds(row0, rows_ps)])
        plsc.subcore_barrier()

        def body(j, carry):
            off = wid * per_w + j * _K
            pltpu.sync_copy(dst_h.at[pl.ds(off, _K)], idxd)
            pltpu.sync_copy(src_h.at[pl.ds(off, _K)], idxs)
            pltpu.async_copy(xu_h.at[idxd], buf, sem).wait()
            pltpu.sync_copy(buf, shared.at[idxs], add=True)
            return carry

        lax.fori_loop(0, steps, body, 0)
        plsc.subcore_barrier()
        pltpu.sync_copy(shared.at[pl.ds(row0, rows_ps)],
                        out_h.at[cid, pl.ds(row0, rows_ps)])

    return kfn(xu, dstx, srcx, zeros)


def _counts(srcx, zeros, ones):
    e = srcx.shape[0]
    n = zeros.shape[0]
    w = zeros.shape[1]
    per_w = e // _NW
    steps = per_w // _K
    rows_ps = n // _NS

    @functools.partial(
        pl.kernel, mesh=_mesh(),
        out_type=jax.ShapeDtypeStruct((_NC, n, w), jnp.float32),
        scratch_types=[pltpu.VMEM((_K,), jnp.int32),
                       pltpu.VMEM((_K, w), jnp.float32),
                       pltpu.VMEM_SHARED((n, w), jnp.float32)])
    def kfn(src_h, zeros_h, ones_h, out_h, idxv, buf, shared):
        cid = lax.axis_index("c")
        sid = lax.axis_index("s")
        wid = sid * _NC + cid
        row0 = sid * rows_ps
        pltpu.sync_copy(zeros_h.at[pl.ds(row0, rows_ps)],
                        shared.at[pl.ds(row0, rows_ps)])
        pltpu.sync_copy(ones_h, buf)
        plsc.subcore_barrier()

        def body(j, carry):
            off = wid * per_w + j * _K
            pltpu.sync_copy(src_h.at[pl.ds(off, _K)], idxv)
            pltpu.sync_copy(buf, shared.at[idxv], add=True)
            return carry

        lax.fori_loop(0, steps, body, 0)
        plsc.subcore_barrier()
        pltpu.sync_copy(shared.at[pl.ds(row0, rows_ps)],
                        out_h.at[cid, pl.ds(row0, rows_ps)])

    return kfn(srcx, zeros, ones)


def _edge_mlp0_body(td, ts, ew1, w2, eb1, eb2, ue_out, nd_out):
    ndv = 0.5 * jnp.abs(td[:, :_D] - ts[:, :_D])
    gsum = 0.5 * (td[:, _D:] + ts[:, _D:])
    bsum = ew1[2 * _D:3 * _D] + ew1[3 * _D:4 * _D]
    h1 = jnp.dot(ndv, bsum, preferred_element_type=jnp.float32) + gsum + eb1[...]
    h2 = jnp.dot(_gelu(h1), w2[...], preferred_element_type=jnp.float32) + eb2[...]
    ue_out[...] = _gelu(h2)
    nd_out[...] = ndv


def _edge_mlp0(td, ts, ew1, w2, eb1, eb2, blk=2000):
    e = td.shape[0]
    grid = e // blk
    row = lambda i: (i, 0)
    full = lambda i: (0, 0)
    return pl.pallas_call(
        _edge_mlp0_body,
        grid=(grid,),
        in_specs=[
            pl.BlockSpec((blk, 2 * _D), row),
            pl.BlockSpec((blk, 2 * _D), row),
            pl.BlockSpec((5 * _D, _D), full),
            pl.BlockSpec((_D, _D), full),
            pl.BlockSpec((1, _D), full),
            pl.BlockSpec((1, _D), full),
        ],
        out_specs=[pl.BlockSpec((blk, _D), row), pl.BlockSpec((blk, _D), row)],
        out_shape=[jax.ShapeDtypeStruct((e, _D), jnp.float32),
                   jax.ShapeDtypeStruct((e, _D), jnp.float32)],
    )(td, ts, ew1, w2, eb1.reshape(1, _D), eb2.reshape(1, _D))


def _edge_mlp_body(td, ts, nd, ew1, w2, eb1, eb2, out):
    absd = 0.5 * jnp.abs(td[:, :_D] - ts[:, :_D])
    gsum = 0.5 * (td[:, _D:] + ts[:, _D:])
    h1 = (jnp.dot(absd, ew1[2 * _D:3 * _D], preferred_element_type=jnp.float32)
          + jnp.dot(nd[...], ew1[3 * _D:4 * _D], preferred_element_type=jnp.float32)
          + gsum + eb1[...])
    h2 = jnp.dot(_gelu(h1), w2[...], preferred_element_type=jnp.float32) + eb2[...]
    out[...] = _gelu(h2)


def _edge_mlp(td, ts, nd, ew1, w2, eb1, eb2, blk=2000):
    e = td.shape[0]
    grid = e // blk
    row = lambda i: (i, 0)
    full = lambda i: (0, 0)
    return pl.pallas_call(
        _edge_mlp_body,
        grid=(grid,),
        in_specs=[
            pl.BlockSpec((blk, 2 * _D), row),
            pl.BlockSpec((blk, 2 * _D), row),
            pl.BlockSpec((blk, _D), row),
            pl.BlockSpec((5 * _D, _D), full),
            pl.BlockSpec((_D, _D), full),
            pl.BlockSpec((1, _D), full),
            pl.BlockSpec((1, _D), full),
        ],
        out_specs=pl.BlockSpec((blk, _D), row),
        out_shape=jax.ShapeDtypeStruct((e, _D), jnp.float32),
    )(td, ts, nd, ew1, w2, eb1.reshape(1, _D), eb2.reshape(1, _D))


def _node_mlp_body(tbl, nodes, sp, cp, w1, w2, nb1, nb2, out):
    x = tbl[:, :_D]
    s = sp[0] + sp[1]
    c = cp[0, :, 0:1] + cp[1, :, 0:1]
    rcc = 1.0 / jnp.maximum(c, 1.0)
    agg = s * rcc
    cat = jnp.concatenate([x, nodes[...], agg], axis=1)
    h1 = jnp.dot(cat, w1[...], preferred_element_type=jnp.float32) + nb1[...]
    h2 = jnp.dot(_gelu(h1), w2[...], preferred_element_type=jnp.float32) + nb2[...]
    out[...] = _gelu(h2)


def _node_mlp(tbl, nodes, sp, cp, w1, w2, nb1, nb2, blk=2000):
    n = nodes.shape[0]
    grid = n // blk
    row = lambda i: (i, 0)
    row3 = lambda i: (0, i, 0)
    full = lambda i: (0, 0)
    return pl.pallas_call(
        _node_mlp_body,
        grid=(grid,),
        in_specs=[
            pl.BlockSpec((blk, 2 * _D), row),
            pl.BlockSpec((blk, _D), row),
            pl.BlockSpec((_NC, blk, _D), row3),
            pl.BlockSpec((_NC, blk, _D), row3),
            pl.BlockSpec((3 * _D, _D), full),
            pl.BlockSpec((_D, _D), full),
            pl.BlockSpec((1, _D), full),
            pl.BlockSpec((1, _D), full),
        ],
        out_specs=pl.BlockSpec((blk, _D), row),
        out_shape=jax.ShapeDtypeStruct((n, _D), jnp.float32),
    )(tbl, nodes, sp, cp, w1, w2, nb1.reshape(1, _D), nb2.reshape(1, _D))


def _finish_body(xu, tp, cp, nodes, ew1n, tbl_out):
    t = tp[0] + tp[1]
    c = cp[0, :, 0:1] + cp[1, :, 0:1]
    rcc = 1.0 / jnp.maximum(c, 1.0)
    xn = 0.5 * (c * xu[...] + t) * rcc
    g = (jnp.dot(xn, ew1n[:_D], preferred_element_type=jnp.float32)
         + jnp.dot(nodes[...], ew1n[_D:2 * _D], preferred_element_type=jnp.float32)
         + jnp.dot(xu[...], ew1n[4 * _D:], preferred_element_type=jnp.float32))
    tbl_out[:, :_D] = xn
    tbl_out[:, _D:] = g


def _finish(xu, tp, cp, nodes, ew1n, blk=2000):
    n = xu.shape[0]
    grid = n // blk
    row = lambda i: (i, 0)
    row3 = lambda i: (0, i, 0)
    full = lambda i: (0, 0)
    return pl.pallas_call(
        _finish_body,
        grid=(grid,),
        in_specs=[
            pl.BlockSpec((blk, _D), row),
            pl.BlockSpec((_NC, blk, _D), row3),
            pl.BlockSpec((_NC, blk, _D), row3),
            pl.BlockSpec((blk, _D), row),
            pl.BlockSpec((5 * _D, _D), full),
        ],
        out_specs=pl.BlockSpec((blk, 2 * _D), row),
        out_shape=jax.ShapeDtypeStruct((n, 2 * _D), jnp.float32),
    )(xu, tp, cp, nodes, ew1n)


def _decode_body(xu, tp, cp, w, b, out):
    t = tp[0] + tp[1]
    c = cp[0, :, 0:1] + cp[1, :, 0:1]
    rcc = 1.0 / jnp.maximum(c, 1.0)
    xn = 0.5 * (c * xu[...] + t) * rcc
    out[...] = jnp.dot(xn, w[...], preferred_element_type=jnp.float32) + b[...]


def _decode(xu, tp, cp, w, b, blk=2000):
    n = xu.shape[0]
    kout = w.shape[1]
    grid = n // blk
    row = lambda i: (i, 0)
    row3 = lambda i: (0, i, 0)
    full = lambda i: (0, 0)
    return pl.pallas_call(
        _decode_body,
        grid=(grid,),
        in_specs=[
            pl.BlockSpec((blk, _D), row),
            pl.BlockSpec((_NC, blk, _D), row3),
            pl.BlockSpec((_NC, blk, _D), row3),
            pl.BlockSpec((_D, kout), full),
            pl.BlockSpec((1, kout), full),
        ],
        out_specs=pl.BlockSpec((blk, kout), row),
        out_shape=jax.ShapeDtypeStruct((n, kout), jnp.float32),
    )(xu, tp, cp, w, b.reshape(1, kout))


def _tables0_body(nodes, ew1, out):
    w = ew1[:_D] + ew1[_D:2 * _D] + ew1[4 * _D:]
    out[:, :_D] = nodes[...]
    out[:, _D:] = jnp.dot(nodes[...], w, preferred_element_type=jnp.float32)


def _tables0(nodes, ew1, blk=2000):
    n = nodes.shape[0]
    grid = n // blk
    return pl.pallas_call(
        _tables0_body,
        grid=(grid,),
        in_specs=[pl.BlockSpec((blk, _D), lambda i: (i, 0)),
                  pl.BlockSpec((5 * _D, _D), lambda i: (0, 0))],
        out_specs=pl.BlockSpec((blk, 2 * _D), lambda i: (i, 0)),
        out_shape=jax.ShapeDtypeStruct((n, 2 * _D), jnp.float32),
    )(nodes, ew1)


def kernel(nodes, params, edge_index):
    n, d = nodes.shape
    L = 3
    src = edge_index[0].astype(jnp.int32)
    dst = edge_index[1].astype(jnp.int32)

    np_ = ((n + _NS * 8 - 1) // (_NS * 8)) * (_NS * 8)
    zeros_n = jnp.zeros((np_, d), jnp.float32)
    ones_k = jnp.ones((_K, d), jnp.float32)

    cp = _counts(src, zeros_n, ones_k)
    tbl = _tables0(nodes, params[0])

    nd = None
    out = None
    for l in range(L):
        eW1, eb1, eW2, eb2, nW1, nb1, nW2, nb2 = params[8 * l:8 * l + 8]

        td, ts = _gather2(tbl, dst, src)
        if l == 0:
            ue, nd = _edge_mlp0(td, ts, eW1, eW2, eb1, eb2)
        else:
            ue = _edge_mlp(td, ts, nd, eW1, eW2, eb1, eb2)

        sp = _scatter_sum(ue, src, zeros_n)
        xu = _node_mlp(tbl, nodes, sp, cp, nW1, nW2, nb1, nb2)

        tp = _smooth_scatter(xu, dst, src, zeros_n)
        if l + 1 < L:
            tbl = _finish(xu, tp, cp, nodes, params[8 * (l + 1)])
        else:
            out = _decode(xu, tp, cp, params[8 * L], params[8 * L + 1])
    return out


# speedup vs baseline: 4.8042x; 1.2094x over previous
import functools
import jax
import jax.numpy as jnp
from jax import lax
from jax.experimental import pallas as pl
from jax.experimental.pallas import tpu as pltpu
from jax.experimental.pallas import tpu_sc as plsc

_D = 128
_NC = 2
_NS = 16
_NW = _NC * _NS
_K = 80
_INV_SQRT2 = 0.7071067811865476


def _gelu(x):
    return 0.5 * x * (1.0 + lax.erf(x * _INV_SQRT2))


def _pack2(x, g):
    xi = jax.lax.bitcast_convert_type(x, jnp.uint32)
    gi = jax.lax.bitcast_convert_type(g, jnp.uint32)
    half = jnp.uint32(0x8000)
    hi = jnp.uint32(0xFFFF0000)
    lo = jnp.uint32(0xFFFF)
    word = ((xi + half) & hi) | (((gi + half) >> 16) & lo)
    return jax.lax.bitcast_convert_type(word, jnp.float32)


def _unpack2(w):
    wi = jax.lax.bitcast_convert_type(w, jnp.uint32)
    x = jax.lax.bitcast_convert_type(wi & jnp.uint32(0xFFFF0000), jnp.float32)
    g = jax.lax.bitcast_convert_type(wi << 16, jnp.float32)
    return x, g


def _mesh():
    return plsc.VectorSubcoreMesh(core_axis_name="c", subcore_axis_name="s")


def _gather2(tbl, dstx, srcx):
    e = dstx.shape[0]
    w = tbl.shape[1]
    per_w = e // _NW
    steps = per_w // _K

    @functools.partial(
        pl.kernel, mesh=_mesh(),
        out_type=[jax.ShapeDtypeStruct((e, w), jnp.float32),
                  jax.ShapeDtypeStruct((e, w), jnp.float32)],
        scratch_types=[pltpu.VMEM((_K,), jnp.int32),
                       pltpu.VMEM((_K,), jnp.int32),
                       pltpu.VMEM((_K, w), jnp.float32),
                       pltpu.VMEM((_K, w), jnp.float32),
                       pltpu.SemaphoreType.DMA,
                       pltpu.SemaphoreType.DMA])
    def kfn(tbl_h, dst_h, src_h, outd_h, outs_h, idxd, idxs, bufd, bufs,
            semd, sems):
        wid = lax.axis_index("s") * _NC + lax.axis_index("c")
        base = wid * per_w

        def body(j, carry):
            off = base + j * _K
            pltpu.sync_copy(dst_h.at[pl.ds(off, _K)], idxd)
            pltpu.sync_copy(src_h.at[pl.ds(off, _K)], idxs)
            cd = pltpu.async_copy(tbl_h.at[idxd], bufd, semd)
            cs = pltpu.async_copy(tbl_h.at[idxs], bufs, sems)
            cd.wait()
            cs.wait()
            pltpu.sync_copy(bufd, outd_h.at[pl.ds(off, _K)])
            pltpu.sync_copy(bufs, outs_h.at[pl.ds(off, _K)])
            return carry

        lax.fori_loop(0, steps, body, 0)

    return kfn(tbl, dstx, srcx)


def _scatter_sum(vals, srcx, zeros):
    e, w = vals.shape
    n = zeros.shape[0]
    per_w = e // _NW
    steps = per_w // _K
    rows_ps = n // _NS

    @functools.partial(
        pl.kernel, mesh=_mesh(),
        out_type=jax.ShapeDtypeStruct((_NC, n, w), jnp.float32),
        scratch_types=[pltpu.VMEM((_K,), jnp.int32),
                       pltpu.VMEM((_K, w), jnp.float32),
                       pltpu.VMEM_SHARED((n, w), jnp.float32)])
    def kfn(vals_h, src_h, zeros_h, out_h, idxv, buf, shared):
        cid = lax.axis_index("c")
        sid = lax.axis_index("s")
        wid = sid * _NC + cid
        row0 = sid * rows_ps
        pltpu.sync_copy(zeros_h.at[pl.ds(row0, rows_ps)],
                        shared.at[pl.ds(row0, rows_ps)])
        plsc.subcore_barrier()

        def body(j, carry):
            off = wid * per_w + j * _K
            pltpu.sync_copy(src_h.at[pl.ds(off, _K)], idxv)
            pltpu.sync_copy(vals_h.at[pl.ds(off, _K)], buf)
            pltpu.sync_copy(buf, shared.at[idxv], add=True)
            return carry

        lax.fori_loop(0, steps, body, 0)
        plsc.subcore_barrier()
        pltpu.sync_copy(shared.at[pl.ds(row0, rows_ps)],
                        out_h.at[cid, pl.ds(row0, rows_ps)])

    return kfn(vals, srcx, zeros)


def _smooth_scatter(xu, dstx, srcx, zeros):
    n, w = zeros.shape
    e = dstx.shape[0]
    per_w = e // _NW
    steps = per_w // _K
    rows_ps = n // _NS

    @functools.partial(
        pl.kernel, mesh=_mesh(),
        out_type=jax.ShapeDtypeStruct((_NC, n, w), jnp.float32),
        scratch_types=[pltpu.VMEM((_K,), jnp.int32),
                       pltpu.VMEM((_K,), jnp.int32),
                       pltpu.VMEM((_K, w), jnp.float32),
                       pltpu.VMEM_SHARED((n, w), jnp.float32),
                       pltpu.SemaphoreType.DMA])
    def kfn(xu_h, dst_h, src_h, zeros_h, out_h, idxd, idxs, buf, shared, sem):
        cid = lax.axis_index("c")
        sid = lax.axis_index("s")
        wid = sid * _NC + cid
        row0 = sid * rows_ps
        pltpu.sync_copy(zeros_h.at[pl.ds(row0, rows_ps)],
                        shared.at[pl.ds(row0, rows_ps)])
        plsc.subcore_barrier()

        def body(j, carry):
            off = wid * per_w + j * _K
            pltpu.sync_copy(dst_h.at[pl.ds(off, _K)], idxd)
            pltpu.sync_copy(src_h.at[pl.ds(off, _K)], idxs)
            pltpu.async_copy(xu_h.at[idxd], buf, sem).wait()
            pltpu.sync_copy(buf, shared.at[idxs], add=True)
            return carry

        lax.fori_loop(0, steps, body, 0)
        plsc.subcore_barrier()
        pltpu.sync_copy(shared.at[pl.ds(row0, rows_ps)],
                        out_h.at[cid, pl.ds(row0, rows_ps)])

    return kfn(xu, dstx, srcx, zeros)


def _counts(srcx, zeros, ones):
    e = srcx.shape[0]
    n = zeros.shape[0]
    w = zeros.shape[1]
    per_w = e // _NW
    steps = per_w // _K
    rows_ps = n // _NS

    @functools.partial(
        pl.kernel, mesh=_mesh(),
        out_type=jax.ShapeDtypeStruct((_NC, n, w), jnp.float32),
        scratch_types=[pltpu.VMEM((_K,), jnp.int32),
                       pltpu.VMEM((_K, w), jnp.float32),
                       pltpu.VMEM_SHARED((n, w), jnp.float32)])
    def kfn(src_h, zeros_h, ones_h, out_h, idxv, buf, shared):
        cid = lax.axis_index("c")
        sid = lax.axis_index("s")
        wid = sid * _NC + cid
        row0 = sid * rows_ps
        pltpu.sync_copy(zeros_h.at[pl.ds(row0, rows_ps)],
                        shared.at[pl.ds(row0, rows_ps)])
        pltpu.sync_copy(ones_h, buf)
        plsc.subcore_barrier()

        def body(j, carry):
            off = wid * per_w + j * _K
            pltpu.sync_copy(src_h.at[pl.ds(off, _K)], idxv)
            pltpu.sync_copy(buf, shared.at[idxv], add=True)
            return carry

        lax.fori_loop(0, steps, body, 0)
        plsc.subcore_barrier()
        pltpu.sync_copy(shared.at[pl.ds(row0, rows_ps)],
                        out_h.at[cid, pl.ds(row0, rows_ps)])

    return kfn(srcx, zeros, ones)


def _edge_mlp0_body(td, ts, ew1, w2, eb1, eb2, ue_out, nd_out):
    xd, gd = _unpack2(td[...])
    xs, gs = _unpack2(ts[...])
    ndv = 0.5 * jnp.abs(xd - xs)
    gsum = 0.5 * (gd + gs)
    bsum = ew1[2 * _D:3 * _D] + ew1[3 * _D:4 * _D]
    h1 = jnp.dot(ndv, bsum, preferred_element_type=jnp.float32) + gsum + eb1[...]
    h2 = jnp.dot(_gelu(h1), w2[...], preferred_element_type=jnp.float32) + eb2[...]
    ue_out[...] = _gelu(h2)
    nd_out[...] = ndv


def _edge_mlp0(td, ts, ew1, w2, eb1, eb2, blk=2000):
    e = td.shape[0]
    grid = e // blk
    row = lambda i: (i, 0)
    full = lambda i: (0, 0)
    return pl.pallas_call(
        _edge_mlp0_body,
        grid=(grid,),
        in_specs=[
            pl.BlockSpec((blk, _D), row),
            pl.BlockSpec((blk, _D), row),
            pl.BlockSpec((5 * _D, _D), full),
            pl.BlockSpec((_D, _D), full),
            pl.BlockSpec((1, _D), full),
            pl.BlockSpec((1, _D), full),
        ],
        out_specs=[pl.BlockSpec((blk, _D), row), pl.BlockSpec((blk, _D), row)],
        out_shape=[jax.ShapeDtypeStruct((e, _D), jnp.float32),
                   jax.ShapeDtypeStruct((e, _D), jnp.float32)],
    )(td, ts, ew1, w2, eb1.reshape(1, _D), eb2.reshape(1, _D))


def _edge_mlp_body(td, ts, nd, ew1, w2, eb1, eb2, out):
    xd, gd = _unpack2(td[...])
    xs, gs = _unpack2(ts[...])
    absd = 0.5 * jnp.abs(xd - xs)
    gsum = 0.5 * (gd + gs)
    h1 = (jnp.dot(absd, ew1[2 * _D:3 * _D], preferred_element_type=jnp.float32)
          + jnp.dot(nd[...], ew1[3 * _D:4 * _D], preferred_element_type=jnp.float32)
          + gsum + eb1[...])
    h2 = jnp.dot(_gelu(h1), w2[...], preferred_element_type=jnp.float32) + eb2[...]
    out[...] = _gelu(h2)


def _edge_mlp(td, ts, nd, ew1, w2, eb1, eb2, blk=2000):
    e = td.shape[0]
    grid = e // blk
    row = lambda i: (i, 0)
    full = lambda i: (0, 0)
    return pl.pallas_call(
        _edge_mlp_body,
        grid=(grid,),
        in_specs=[
            pl.BlockSpec((blk, _D), row),
            pl.BlockSpec((blk, _D), row),
            pl.BlockSpec((blk, _D), row),
            pl.BlockSpec((5 * _D, _D), full),
            pl.BlockSpec((_D, _D), full),
            pl.BlockSpec((1, _D), full),
            pl.BlockSpec((1, _D), full),
        ],
        out_specs=pl.BlockSpec((blk, _D), row),
        out_shape=jax.ShapeDtypeStruct((e, _D), jnp.float32),
    )(td, ts, nd, ew1, w2, eb1.reshape(1, _D), eb2.reshape(1, _D))


def _node_mlp_body(x_in, nodes, sp, cp, w1, w2, nb1, nb2, out):
    x = x_in[...]
    s = sp[0] + sp[1]
    c = cp[0, :, 0:1] + cp[1, :, 0:1]
    rcc = 1.0 / jnp.maximum(c, 1.0)
    agg = s * rcc
    cat = jnp.concatenate([x, nodes[...], agg], axis=1)
    h1 = jnp.dot(cat, w1[...], preferred_element_type=jnp.float32) + nb1[...]
    h2 = jnp.dot(_gelu(h1), w2[...], preferred_element_type=jnp.float32) + nb2[...]
    out[...] = _gelu(h2)


def _node_mlp(x_in, nodes, sp, cp, w1, w2, nb1, nb2, blk=2000):
    n = nodes.shape[0]
    grid = n // blk
    row = lambda i: (i, 0)
    row3 = lambda i: (0, i, 0)
    full = lambda i: (0, 0)
    return pl.pallas_call(
        _node_mlp_body,
        grid=(grid,),
        in_specs=[
            pl.BlockSpec((blk, _D), row),
            pl.BlockSpec((blk, _D), row),
            pl.BlockSpec((_NC, blk, _D), row3),
            pl.BlockSpec((_NC, blk, _D), row3),
            pl.BlockSpec((3 * _D, _D), full),
            pl.BlockSpec((_D, _D), full),
            pl.BlockSpec((1, _D), full),
            pl.BlockSpec((1, _D), full),
        ],
        out_specs=pl.BlockSpec((blk, _D), row),
        out_shape=jax.ShapeDtypeStruct((n, _D), jnp.float32),
    )(x_in, nodes, sp, cp, w1, w2, nb1.reshape(1, _D), nb2.reshape(1, _D))


def _finish_body(xu, tp, cp, nodes, ew1n, tbl_out, x_out):
    t = tp[0] + tp[1]
    c = cp[0, :, 0:1] + cp[1, :, 0:1]
    rcc = 1.0 / jnp.maximum(c, 1.0)
    xn = 0.5 * (c * xu[...] + t) * rcc
    g = (jnp.dot(xn, ew1n[:_D], preferred_element_type=jnp.float32)
         + jnp.dot(nodes[...], ew1n[_D:2 * _D], preferred_element_type=jnp.float32)
         + jnp.dot(xu[...], ew1n[4 * _D:], preferred_element_type=jnp.float32))
    tbl_out[...] = _pack2(xn, g)
    x_out[...] = xn


def _finish(xu, tp, cp, nodes, ew1n, blk=2000):
    n = xu.shape[0]
    grid = n // blk
    row = lambda i: (i, 0)
    row3 = lambda i: (0, i, 0)
    full = lambda i: (0, 0)
    return pl.pallas_call(
        _finish_body,
        grid=(grid,),
        in_specs=[
            pl.BlockSpec((blk, _D), row),
            pl.BlockSpec((_NC, blk, _D), row3),
            pl.BlockSpec((_NC, blk, _D), row3),
            pl.BlockSpec((blk, _D), row),
            pl.BlockSpec((5 * _D, _D), full),
        ],
        out_specs=[pl.BlockSpec((blk, _D), row), pl.BlockSpec((blk, _D), row)],
        out_shape=[jax.ShapeDtypeStruct((n, _D), jnp.float32),
                   jax.ShapeDtypeStruct((n, _D), jnp.float32)],
    )(xu, tp, cp, nodes, ew1n)


def _decode_body(xu, tp, cp, w, b, out):
    t = tp[0] + tp[1]
    c = cp[0, :, 0:1] + cp[1, :, 0:1]
    rcc = 1.0 / jnp.maximum(c, 1.0)
    xn = 0.5 * (c * xu[...] + t) * rcc
    out[...] = jnp.dot(xn, w[...], preferred_element_type=jnp.float32) + b[...]


def _decode(xu, tp, cp, w, b, blk=2000):
    n = xu.shape[0]
    kout = w.shape[1]
    grid = n // blk
    row = lambda i: (i, 0)
    row3 = lambda i: (0, i, 0)
    full = lambda i: (0, 0)
    return pl.pallas_call(
        _decode_body,
        grid=(grid,),
        in_specs=[
            pl.BlockSpec((blk, _D), row),
            pl.BlockSpec((_NC, blk, _D), row3),
            pl.BlockSpec((_NC, blk, _D), row3),
            pl.BlockSpec((_D, kout), full),
            pl.BlockSpec((1, kout), full),
        ],
        out_specs=pl.BlockSpec((blk, kout), row),
        out_shape=jax.ShapeDtypeStruct((n, kout), jnp.float32),
    )(xu, tp, cp, w, b.reshape(1, kout))


def _tables0_body(nodes, ew1, out):
    w = ew1[:_D] + ew1[_D:2 * _D] + ew1[4 * _D:]
    g = jnp.dot(nodes[...], w, preferred_element_type=jnp.float32)
    out[...] = _pack2(nodes[...], g)


def _tables0(nodes, ew1, blk=2000):
    n = nodes.shape[0]
    grid = n // blk
    return pl.pallas_call(
        _tables0_body,
        grid=(grid,),
        in_specs=[pl.BlockSpec((blk, _D), lambda i: (i, 0)),
                  pl.BlockSpec((5 * _D, _D), lambda i: (0, 0))],
        out_specs=pl.BlockSpec((blk, _D), lambda i: (i, 0)),
        out_shape=jax.ShapeDtypeStruct((n, _D), jnp.float32),
    )(nodes, ew1)


def kernel(nodes, params, edge_index):
    n, d = nodes.shape
    L = 3
    src = edge_index[0].astype(jnp.int32)
    dst = edge_index[1].astype(jnp.int32)

    np_ = ((n + _NS * 8 - 1) // (_NS * 8)) * (_NS * 8)
    zeros_n = jnp.zeros((np_, d), jnp.float32)
    ones_k = jnp.ones((_K, d), jnp.float32)

    cp = _counts(src, zeros_n, ones_k)
    tbl = _tables0(nodes, params[0])
    x = nodes

    nd = None
    out = None
    for l in range(L):
        eW1, eb1, eW2, eb2, nW1, nb1, nW2, nb2 = params[8 * l:8 * l + 8]

        td, ts = _gather2(tbl, dst, src)
        if l == 0:
            ue, nd = _edge_mlp0(td, ts, eW1, eW2, eb1, eb2)
        else:
            ue = _edge_mlp(td, ts, nd, eW1, eW2, eb1, eb2)

        sp = _scatter_sum(ue, src, zeros_n)
        xu = _node_mlp(x, nodes, sp, cp, nW1, nW2, nb1, nb2)

        tp = _smooth_scatter(xu, dst, src, zeros_n)
        if l + 1 < L:
            tbl, x = _finish(xu, tp, cp, nodes, params[8 * (l + 1)])
        else:
            out = _decode(xu, tp, cp, params[8 * L], params[8 * L + 1])
    return out
